# baseline (speedup 1.0000x reference)
import functools

import jax
import jax.numpy as jnp
from jax import lax
from jax.experimental import pallas as pl
from jax.experimental.pallas import tpu as pltpu
from jax.experimental.pallas import tpu_sc as plsc

VOCAB = 1000000
EMBED_DIM = 32
BATCH = 4096
HIST = 20
IN_FEAT = HIST * EMBED_DIM

NUM_WORKERS = 32
BPW = BATCH // NUM_WORKERS
NBUF = 10
NPRE = 5

_mesh = plsc.VectorSubcoreMesh(core_axis_name="c", subcore_axis_name="s")


@functools.partial(
    pl.kernel,
    mesh=_mesh,
    out_type=jax.ShapeDtypeStruct((HIST, BATCH, EMBED_DIM), jnp.int32),
    scratch_types=[
        pltpu.VMEM((HIST, BPW), jnp.int32),
        pltpu.VMEM((NBUF, BPW, EMBED_DIM), jnp.int32),
        pltpu.SemaphoreType.DMA,
        pltpu.SemaphoreType.DMA,
    ],
    compiler_params=pltpu.CompilerParams(use_tc_tiling_on_sc=False),
)
def _sc_gather(tq_hbm, xq_hbm, out_hbm, idx_v, bufs, gsem, wsem):
    wid = lax.axis_index("s") * 2 + lax.axis_index("c")
    pltpu.sync_copy(xq_hbm.at[wid], idx_v)
    gops = [None] * HIST
    wops = [None] * HIST
    for p in range(NPRE):
        gops[p] = pltpu.async_copy(
            tq_hbm.at[idx_v.at[p]], bufs.at[p % NBUF], gsem)
    for p in range(HIST):
        gops[p].wait()
        wops[p] = pltpu.async_copy(
            bufs.at[p % NBUF], out_hbm.at[p, pl.ds(wid * BPW, BPW)], wsem)
        nxt = p + NPRE
        if nxt < HIST:
            if nxt - NBUF >= 0:
                wops[nxt - NBUF].wait()
            gops[nxt] = pltpu.async_copy(
                tq_hbm.at[idx_v.at[nxt]], bufs.at[nxt % NBUF], gsem)
    for p in range(HIST - NBUF, HIST):
        wops[p].wait()


QSTRIDE = 1 << 17
_TR = 4096


def _tr_body(*refs):
    o_ref = refs[-1]
    tq4 = _TR // 4
    rows = []
    for q in range(4):
        lo = refs[2 * q][...].astype(jnp.bfloat16).astype(jnp.float32)
        hi = refs[2 * q + 1][...].astype(jnp.bfloat16).astype(jnp.float32)
        lo_i = lax.bitcast_convert_type(lo, jnp.int32)
        hi_i = lax.bitcast_convert_type(hi, jnp.int32)
        packed = (hi_i & jnp.int32(-65536)) | (
            (lo_i >> 16) & jnp.int32(0xFFFF))
        y = jnp.concatenate(
            [packed[:, c * tq4:(c + 1) * tq4] for c in range(4)], axis=0)
        rows.append(y.T)
    o_ref[...] = jnp.concatenate(rows, axis=0)


def _tc_transpose(tt):
    nb = QSTRIDE // _TR
    last = (VOCAB - 1) // _TR
    specs = [
        pl.BlockSpec(
            (32, _TR),
            (lambda g: (lambda i: (0, jnp.minimum(g * nb + i, last))))(g))
        for g in range(8)
    ]
    return pl.pallas_call(
        _tr_body,
        grid=(nb,),
        in_specs=specs,
        out_specs=pl.BlockSpec((_TR, 128), lambda i: (i, 0)),
        out_shape=jax.ShapeDtypeStruct((QSTRIDE, 128), jnp.int32),
    )(*([tt] * 8))


_BB = 512


def _mlp_body(cand_ref, sel_ref, w1_ref, b1_ref, w2_ref, b2_ref, w3_ref,
              b3_ref, o_ref):
    fb = _BB // 4
    grp = lax.broadcasted_iota(jnp.int32, (fb, 128), 1) >> 5
    acc = jnp.zeros((fb, 512), jnp.float32)
    for p in range(HIST):
        c = cand_ref[p]
        w = sel_ref[:, p:p + 1]
        pm = (w >> grp) & 1
        bits = jnp.where(pm == 1, c & jnp.int32(-65536), c << 16)
        v = lax.bitcast_convert_type(bits, jnp.float32).astype(jnp.bfloat16)
        acc = acc + jnp.dot(v, w1_ref[p], preferred_element_type=jnp.float32)
    a1 = jnp.maximum(acc + b1_ref[...], 0.0).astype(jnp.bfloat16)
    a2 = jnp.maximum(
        jnp.dot(a1, w2_ref[...], preferred_element_type=jnp.float32)
        + b2_ref[...], 0.0).astype(jnp.bfloat16)
    o_ref[...] = (
        jnp.dot(a2, w3_ref[...], preferred_element_type=jnp.float32)
        + b3_ref[...])


def _tc_mlp(cand, self_, W1bd, b1f, W2bd, b2f, W3bd, b3f):
    fb = _BB // 4
    return pl.pallas_call(
        _mlp_body,
        grid=(BATCH // _BB,),
        in_specs=[
            pl.BlockSpec((HIST, fb, 128), lambda i: (0, i, 0)),
            pl.BlockSpec((fb, HIST), lambda i: (i, 0)),
            pl.BlockSpec((HIST, 128, 512), lambda i: (0, 0, 0)),
            pl.BlockSpec((1, 512), lambda i: (0, 0)),
            pl.BlockSpec((512, 256), lambda i: (0, 0)),
            pl.BlockSpec((1, 256), lambda i: (0, 0)),
            pl.BlockSpec((256, 512), lambda i: (0, 0)),
            pl.BlockSpec((1, 512), lambda i: (0, 0)),
        ],
        out_specs=pl.BlockSpec((fb, 512), lambda i: (i, 0)),
        out_shape=jax.ShapeDtypeStruct((BATCH // 4, 512), jnp.float32),
    )(cand, self_, W1bd, b1f, W2bd, b2f, W3bd, b3f)


def kernel(x, table, W1, b1, W2, b2, W3, b3):
    xi = x.astype(jnp.int32)
    tq = _tc_transpose(table.T)
    tqv = tq.reshape(4 * QSTRIDE, EMBED_DIM)
    R = xi & (QSTRIDE - 1)
    qg = xi >> 18
    r128 = (R & jnp.int32(-4096)) | (qg << 10) | (R & jnp.int32(1023))
    cgrp = (R >> 10) & 3
    xq = (r128 << 2) | cgrp
    par = (xi >> 17) & 1
    xq_t = xq.reshape(NUM_WORKERS, BPW, HIST).transpose(0, 2, 1)
    g = _sc_gather(tqv, xq_t)
    gv = g.reshape(HIST, BATCH // 4, 128)
    par4 = par.reshape(BATCH // 4, 4, HIST)
    sel_f = jnp.sum(par4 << jnp.array([0, 1, 2, 3])[None, :, None], axis=1)
    W13 = W1.reshape(HIST, EMBED_DIM, 128)
    W3p = jnp.pad(W3, ((0, 0), (0, 124)))
    b3p = jnp.pad(b3, (0, 124))
    eye4 = jnp.eye(4, dtype=jnp.float32)
    W1bd = jnp.einsum("ab,pdf->padbf", eye4, W13).reshape(
        HIST, 128, 512).astype(jnp.bfloat16)
    W2bd = jnp.einsum("ab,df->adbf", eye4, W2).reshape(
        512, 256).astype(jnp.bfloat16)
    W3bd = jnp.einsum("ab,df->adbf", eye4, W3p).reshape(
        256, 512).astype(jnp.bfloat16)
    b1f = jnp.tile(b1, 4).reshape(1, 512)
    b2f = jnp.tile(b2, 4).reshape(1, 256)
    b3f = jnp.tile(b3p, 4).reshape(1, 512)
    outf = _tc_mlp(gv, sel_f, W1bd, b1f, W2bd, b2f, W3bd, b3f)
    return outf.reshape(BATCH, 128)[:, :4]


# speedup vs baseline: 1.0363x; 1.0363x over previous
import functools

import jax
import jax.numpy as jnp
from jax import lax
from jax.experimental import pallas as pl
from jax.experimental.pallas import tpu as pltpu
from jax.experimental.pallas import tpu_sc as plsc

VOCAB = 1000000
EMBED_DIM = 32
BATCH = 4096
HIST = 20
IN_FEAT = HIST * EMBED_DIM

NUM_WORKERS = 32
BPW = BATCH // NUM_WORKERS
NBUF = 6
NPRE = 3

_mesh = plsc.VectorSubcoreMesh(core_axis_name="c", subcore_axis_name="s")


@functools.partial(
    pl.kernel,
    mesh=_mesh,
    out_type=jax.ShapeDtypeStruct((HIST, BATCH, EMBED_DIM), jnp.int32),
    scratch_types=[
        pltpu.VMEM((HIST, BPW), jnp.int32),
        pltpu.VMEM((NBUF, BPW, EMBED_DIM), jnp.int32),
        pltpu.SemaphoreType.DMA,
        pltpu.SemaphoreType.DMA,
    ],
    compiler_params=pltpu.CompilerParams(use_tc_tiling_on_sc=False),
)
def _sc_gather(tq_hbm, xq_hbm, out_hbm, idx_v, bufs, gsem, wsem):
    wid = lax.axis_index("s") * 2 + lax.axis_index("c")
    pltpu.sync_copy(xq_hbm.at[wid], idx_v)
    gops = [None] * HIST
    wops = [None] * HIST
    for p in range(NPRE):
        gops[p] = pltpu.async_copy(
            tq_hbm.at[idx_v.at[p]], bufs.at[p % NBUF], gsem)
    for p in range(HIST):
        gops[p].wait()
        wops[p] = pltpu.async_copy(
            bufs.at[p % NBUF], out_hbm.at[p, pl.ds(wid * BPW, BPW)], wsem)
        nxt = p + NPRE
        if nxt < HIST:
            if nxt - NBUF >= 0:
                wops[nxt - NBUF].wait()
            gops[nxt] = pltpu.async_copy(
                tq_hbm.at[idx_v.at[nxt]], bufs.at[nxt % NBUF], gsem)
    for p in range(HIST - NBUF, HIST):
        wops[p].wait()


QSTRIDE = 1 << 17
_TR = 8192


def _tr_body(*refs):
    o_ref = refs[-1]
    tq4 = _TR // 4
    rows = []
    for q in range(4):
        lo = refs[2 * q][...].astype(jnp.bfloat16).astype(jnp.float32)
        hi = refs[2 * q + 1][...].astype(jnp.bfloat16).astype(jnp.float32)
        lo_i = lax.bitcast_convert_type(lo, jnp.int32)
        hi_i = lax.bitcast_convert_type(hi, jnp.int32)
        packed = (hi_i & jnp.int32(-65536)) | (
            (lo_i >> 16) & jnp.int32(0xFFFF))
        y = jnp.concatenate(
            [packed[:, c * tq4:(c + 1) * tq4] for c in range(4)], axis=0)
        rows.append(y.T)
    o_ref[...] = jnp.concatenate(rows, axis=0)


def _tc_transpose(tt):
    nb = QSTRIDE // _TR
    last = (VOCAB - 1) // _TR
    specs = [
        pl.BlockSpec(
            (32, _TR),
            (lambda g: (lambda i: (0, jnp.minimum(g * nb + i, last))))(g))
        for g in range(8)
    ]
    return pl.pallas_call(
        _tr_body,
        grid=(nb,),
        in_specs=specs,
        out_specs=pl.BlockSpec((_TR, 128), lambda i: (i, 0)),
        out_shape=jax.ShapeDtypeStruct((QSTRIDE, 128), jnp.int32),
    )(*([tt] * 8))


_BB = 512


def _mlp_body(cand_ref, sel_ref, w1_ref, b1_ref, w2_ref, b2_ref, w3_ref,
              b3_ref, o_ref):
    fb = _BB // 4
    grp = lax.broadcasted_iota(jnp.int32, (fb, 128), 1) >> 5
    acc = jnp.zeros((fb, 512), jnp.float32)
    for p in range(HIST):
        c = cand_ref[p]
        w = sel_ref[:, p:p + 1]
        pm = (w >> grp) & 1
        bits = jnp.where(pm == 1, c & jnp.int32(-65536), c << 16)
        v = lax.bitcast_convert_type(bits, jnp.float32).astype(jnp.bfloat16)
        acc = acc + jnp.dot(v, w1_ref[p], preferred_element_type=jnp.float32)
    a1 = jnp.maximum(acc + b1_ref[...], 0.0).astype(jnp.bfloat16)
    a2 = jnp.maximum(
        jnp.dot(a1, w2_ref[...], preferred_element_type=jnp.float32)
        + b2_ref[...], 0.0).astype(jnp.bfloat16)
    o_ref[...] = (
        jnp.dot(a2, w3_ref[...], preferred_element_type=jnp.float32)
        + b3_ref[...])


def _tc_mlp(cand, self_, W1bd, b1f, W2bd, b2f, W3bd, b3f):
    fb = _BB // 4
    return pl.pallas_call(
        _mlp_body,
        grid=(BATCH // _BB,),
        in_specs=[
            pl.BlockSpec((HIST, fb, 128), lambda i: (0, i, 0)),
            pl.BlockSpec((fb, HIST), lambda i: (i, 0)),
            pl.BlockSpec((HIST, 128, 512), lambda i: (0, 0, 0)),
            pl.BlockSpec((1, 512), lambda i: (0, 0)),
            pl.BlockSpec((512, 256), lambda i: (0, 0)),
            pl.BlockSpec((1, 256), lambda i: (0, 0)),
            pl.BlockSpec((256, 512), lambda i: (0, 0)),
            pl.BlockSpec((1, 512), lambda i: (0, 0)),
        ],
        out_specs=pl.BlockSpec((fb, 512), lambda i: (i, 0)),
        out_shape=jax.ShapeDtypeStruct((BATCH // 4, 512), jnp.float32),
    )(cand, self_, W1bd, b1f, W2bd, b2f, W3bd, b3f)


def kernel(x, table, W1, b1, W2, b2, W3, b3):
    xi = x.astype(jnp.int32)
    tq = _tc_transpose(table.T)
    tqv = tq.reshape(4 * QSTRIDE, EMBED_DIM)
    R = xi & (QSTRIDE - 1)
    qg = xi >> 18
    r128 = (R & jnp.int32(-4096)) | (qg << 10) | (R & jnp.int32(1023))
    cgrp = (R >> 10) & 3
    xq = (r128 << 2) | cgrp
    par = (xi >> 17) & 1
    xq_t = xq.reshape(NUM_WORKERS, BPW, HIST).transpose(0, 2, 1)
    g = _sc_gather(tqv, xq_t)
    gv = g.reshape(HIST, BATCH // 4, 128)
    par4 = par.reshape(BATCH // 4, 4, HIST)
    sel_f = jnp.sum(par4 << jnp.array([0, 1, 2, 3])[None, :, None], axis=1)
    W13 = W1.reshape(HIST, EMBED_DIM, 128)
    W3p = jnp.pad(W3, ((0, 0), (0, 124)))
    b3p = jnp.pad(b3, (0, 124))
    eye4 = jnp.eye(4, dtype=jnp.float32)
    W1bd = jnp.einsum("ab,pdf->padbf", eye4, W13).reshape(
        HIST, 128, 512).astype(jnp.bfloat16)
    W2bd = jnp.einsum("ab,df->adbf", eye4, W2).reshape(
        512, 256).astype(jnp.bfloat16)
    W3bd = jnp.einsum("ab,df->adbf", eye4, W3p).reshape(
        256, 512).astype(jnp.bfloat16)
    b1f = jnp.tile(b1, 4).reshape(1, 512)
    b2f = jnp.tile(b2, 4).reshape(1, 256)
    b3f = jnp.tile(b3p, 4).reshape(1, 512)
    outf = _tc_mlp(gv, sel_f, W1bd, b1f, W2bd, b2f, W3bd, b3f)
    return outf.reshape(BATCH, 128)[:, :4]
